# baseline (speedup 1.0000x reference)
import functools

import jax
import jax.numpy as jnp
from jax import lax
from jax.experimental import pallas as pl
from jax.experimental.pallas import tpu as pltpu
from jax.experimental.pallas import tpu_sc as plsc

B, S, D = 16, 2048, 768
L = 16
C = 32
NB = 5
RA = 3
NCHUNK = S // C
WPB = 2
NVEC = S // L
DL = D // L
MCH = NCHUNK // WPB


def _compact_body(x_hbm, valid_hbm, out_hbm, valid_v, srcidx_v,
                  gb0, gb1, gb2, gb3, gb4, shared_z,
                  sg0, sg1, sg2, sg3, sg4, sw0, sw1, sw2, sw3, sw4, sz):
    cid = lax.axis_index("c")
    sid = lax.axis_index("s")
    b = sid
    sub = (sid + cid) % WPB
    base = b * S
    gbufs = (gb0, gb1, gb2, gb3, gb4)
    sgs = (sg0, sg1, sg2, sg3, sg4)
    sws = (sw0, sw1, sw2, sw3, sw4)

    pltpu.sync_copy(valid_hbm.at[b], valid_v)

    @pl.when(sid == 0)
    def _():
        def _zrow(r, carry):
            for i in range(DL):
                gb0[r, pl.ds(i * L, L)] = jnp.zeros((L,), jnp.float32)
            return carry

        lax.fori_loop(0, C, _zrow, 0)
        pltpu.sync_copy(gb0, shared_z)

    def _cnt(i, carry):
        m0 = valid_v[pl.ds((2 * i) * L, L)] == 1
        m1 = valid_v[pl.ds((2 * i + 1) * L, L)] == 1
        return (carry + plsc.all_reduce_population_count(m0)
                + plsc.all_reduce_population_count(m1))

    count_vec = lax.fori_loop(0, NVEC // 2, _cnt, jnp.zeros((L,), jnp.int32))
    count = count_vec[0]

    n_full = count // C
    rem = count - n_full * C
    ng_all = (count + C - 1) // C
    mg = jnp.maximum(0, (ng_all - sub + 1) // WPB)

    plsc.subcore_barrier()

    def _zfire(i, carry):
        j0 = base + (sub + WPB * i) * C
        pltpu.async_copy(shared_z, out_hbm.at[pl.ds(j0, C)], sz)
        return carry

    lax.fori_loop(mg, MCH, _zfire, 0)

    def _init(i, carry):
        srcidx_v[pl.ds(i * L, L)] = jnp.full((L,), base, jnp.int32)
        return carry

    lax.fori_loop(0, NVEC, _init, 0)

    lanes = lax.iota(jnp.int32, L)

    def _prefix(i, carry):
        v0 = valid_v[pl.ds((2 * i) * L, L)]
        m0 = v0 == 1
        incl0 = plsc.cumsum(v0)
        v1 = valid_v[pl.ds((2 * i + 1) * L, L)]
        m1 = v1 == 1
        incl1 = plsc.cumsum(v1)
        plsc.store_scatter(
            srcidx_v, [incl0 - v0 + carry], base + (2 * i) * L + lanes, mask=m0
        )
        c1 = carry + plsc.all_reduce_population_count(m0)
        plsc.store_scatter(
            srcidx_v, [incl1 - v1 + c1], base + (2 * i + 1) * L + lanes, mask=m1
        )
        return c1 + plsc.all_reduce_population_count(m1)

    lax.fori_loop(0, NVEC // 2, _prefix, jnp.zeros((L,), jnp.int32))

    def _start_g(i, buf, sem):
        k = sub + WPB * i
        pltpu.async_copy(x_hbm.at[srcidx_v.at[pl.ds(k * C, C)]], buf, sem)

    def _wait(buf, sem):
        pltpu.make_async_copy(x_hbm.at[pl.ds(0, C)], buf, sem).wait()

    def _finish_g(i, buf, wsem):
        k = sub + WPB * i

        @pl.when(jnp.logical_and(k == n_full, rem > 0))
        def _():
            def _ztail(r, c2):
                for q in range(DL):
                    buf[r, pl.ds(q * L, L)] = jnp.zeros((L,), jnp.float32)
                return c2

            lax.fori_loop(rem, C, _ztail, 0)

        pltpu.async_copy(buf, out_hbm.at[pl.ds(base + k * C, C)], wsem)

    for p in range(RA):
        @pl.when(mg > p)
        def _(p=p):
            _start_g(p, gbufs[p], sgs[p])

    def _quad(q, carry):
        i0 = NB * q
        for j in range(NB):
            i = i0 + j
            ja = (j + RA) % NB

            @pl.when(i < mg)
            def _(i=i, j=j, ja=ja):
                @pl.when(i + RA < mg)
                def _():
                    @pl.when(i >= 2)
                    def _():
                        _wait(gbufs[ja], sws[ja])

                    _start_g(i + RA, gbufs[ja], sgs[ja])

                _wait(gbufs[j], sgs[j])
                _finish_g(i, gbufs[j], sws[j])

        return carry

    lax.fori_loop(0, -(-MCH // NB), _quad, 0)

    for j in range(NB):
        nfired = jnp.maximum(0, (mg + (NB - 1) - j) // NB)
        nwaited = jnp.maximum(0, (mg - 1 - j) // NB)

        def _wdrain(i, carry, j=j):
            _wait(gbufs[j], sws[j])
            return carry

        lax.fori_loop(0, nfired - nwaited, _wdrain, 0)

    def _zdrain(i, carry):
        pltpu.make_async_copy(x_hbm.at[pl.ds(0, C)], shared_z, sz).wait()
        return carry

    lax.fori_loop(0, MCH - mg, _zdrain, 0)


@functools.partial(jax.jit, static_argnums=())
def _compact(x_flat, valid_ids):
    mesh = plsc.VectorSubcoreMesh(core_axis_name="c", subcore_axis_name="s")
    f = pl.kernel(
        _compact_body,
        out_type=jax.ShapeDtypeStruct((B * S, D), jnp.float32),
        mesh=mesh,
        compiler_params=pltpu.CompilerParams(needs_layout_passes=False),
        scratch_types=[
            pltpu.VMEM((S,), jnp.int32),
            pltpu.VMEM((S,), jnp.int32),
            pltpu.VMEM((C, D), jnp.float32),
            pltpu.VMEM((C, D), jnp.float32),
            pltpu.VMEM((C, D), jnp.float32),
            pltpu.VMEM((C, D), jnp.float32),
            pltpu.VMEM((C, D), jnp.float32),
            pltpu.VMEM_SHARED((C, D), jnp.float32),
            pltpu.SemaphoreType.DMA,
            pltpu.SemaphoreType.DMA,
            pltpu.SemaphoreType.DMA,
            pltpu.SemaphoreType.DMA,
            pltpu.SemaphoreType.DMA,
            pltpu.SemaphoreType.DMA,
            pltpu.SemaphoreType.DMA,
            pltpu.SemaphoreType.DMA,
            pltpu.SemaphoreType.DMA,
            pltpu.SemaphoreType.DMA,
            pltpu.SemaphoreType.DMA,
        ],
    )
    return f(x_flat, valid_ids)


def kernel(sequence_output, valid_ids):
    x_flat = sequence_output.reshape(B * S, D)
    out = _compact(x_flat, valid_ids)
    return out.reshape(B, S, D)


# speedup vs baseline: 1.0120x; 1.0120x over previous
import functools

import jax
import jax.numpy as jnp
from jax import lax
from jax.experimental import pallas as pl
from jax.experimental.pallas import tpu as pltpu
from jax.experimental.pallas import tpu_sc as plsc

B, S, D = 16, 2048, 768
L = 16
C = 32
NB = 4
NCHUNK = S // C
WPB = 2
NVEC = S // L
DL = D // L
MCH = NCHUNK // WPB


def _compact_body(x_hbm, valid_hbm, out_hbm, valid_v, srcidx_v,
                  gb0, gb1, gb2, gb3, shared_z,
                  sg0, sg1, sg2, sg3, sw0, sw1, sw2, sw3, sz):
    cid = lax.axis_index("c")
    sid = lax.axis_index("s")
    b = sid
    sub = (sid + cid) % WPB
    base = b * S
    gbufs = (gb0, gb1, gb2, gb3)
    sgs = (sg0, sg1, sg2, sg3)
    sws = (sw0, sw1, sw2, sw3)

    pltpu.sync_copy(valid_hbm.at[b], valid_v)

    @pl.when(sid == 0)
    def _():
        def _zrow(r, carry):
            for i in range(DL):
                gb0[r, pl.ds(i * L, L)] = jnp.zeros((L,), jnp.float32)
            return carry

        lax.fori_loop(0, C, _zrow, 0)
        pltpu.sync_copy(gb0, shared_z)

    def _cnt(i, carry):
        m0 = valid_v[pl.ds((2 * i) * L, L)] == 1
        m1 = valid_v[pl.ds((2 * i + 1) * L, L)] == 1
        return (carry + plsc.all_reduce_population_count(m0)
                + plsc.all_reduce_population_count(m1))

    count_vec = lax.fori_loop(0, NVEC // 2, _cnt, jnp.zeros((L,), jnp.int32))
    count = count_vec[0]

    n_full = count // C
    rem = count - n_full * C
    ng_all = (count + C - 1) // C
    mg = jnp.maximum(0, (ng_all - sub + 1) // WPB)

    plsc.subcore_barrier()

    def _zfire(i, carry):
        j0 = base + (sub + WPB * i) * C
        pltpu.async_copy(shared_z, out_hbm.at[pl.ds(j0, C)], sz)
        return carry

    lax.fori_loop(mg, MCH, _zfire, 0)

    def _init(i, carry):
        srcidx_v[pl.ds(i * L, L)] = jnp.full((L,), base, jnp.int32)
        return carry

    lax.fori_loop(0, NVEC, _init, 0)

    lanes = lax.iota(jnp.int32, L)

    def _prefix(i, carry):
        v0 = valid_v[pl.ds((2 * i) * L, L)]
        m0 = v0 == 1
        incl0 = plsc.cumsum(v0)
        v1 = valid_v[pl.ds((2 * i + 1) * L, L)]
        m1 = v1 == 1
        incl1 = plsc.cumsum(v1)
        plsc.store_scatter(
            srcidx_v, [incl0 - v0 + carry], base + (2 * i) * L + lanes, mask=m0
        )
        c1 = carry + plsc.all_reduce_population_count(m0)
        plsc.store_scatter(
            srcidx_v, [incl1 - v1 + c1], base + (2 * i + 1) * L + lanes, mask=m1
        )
        return c1 + plsc.all_reduce_population_count(m1)

    lax.fori_loop(0, NVEC // 2, _prefix, jnp.zeros((L,), jnp.int32))

    def _start_g(i, buf, sem):
        k = sub + WPB * i
        pltpu.async_copy(x_hbm.at[srcidx_v.at[pl.ds(k * C, C)]], buf, sem)

    def _wait(buf, sem):
        pltpu.make_async_copy(x_hbm.at[pl.ds(0, C)], buf, sem).wait()

    def _finish_g(i, buf, wsem):
        k = sub + WPB * i

        @pl.when(jnp.logical_and(k == n_full, rem > 0))
        def _():
            def _ztail(r, c2):
                for q in range(DL):
                    buf[r, pl.ds(q * L, L)] = jnp.zeros((L,), jnp.float32)
                return c2

            lax.fori_loop(rem, C, _ztail, 0)

        pltpu.async_copy(buf, out_hbm.at[pl.ds(base + k * C, C)], wsem)

    @pl.when(mg > 0)
    def _():
        _start_g(0, gb0, sg0)

    @pl.when(mg > 1)
    def _():
        _start_g(1, gb1, sg1)

    def _quad(q, carry):
        i0 = NB * q
        for j in range(NB):
            i = i0 + j
            ja = (j + 2) % NB

            @pl.when(i < mg)
            def _(i=i, j=j, ja=ja):
                @pl.when(i + 2 < mg)
                def _():
                    @pl.when(i >= 2)
                    def _():
                        _wait(gbufs[ja], sws[ja])

                    _start_g(i + 2, gbufs[ja], sgs[ja])

                _wait(gbufs[j], sgs[j])
                _finish_g(i, gbufs[j], sws[j])

        return carry

    lax.fori_loop(0, MCH // NB, _quad, 0)

    for j in range(NB):
        nfired = jnp.maximum(0, (mg + (NB - 1) - j) // NB)
        nwaited = jnp.maximum(0, (mg - 1 - j) // NB)

        def _wdrain(i, carry, j=j):
            _wait(gbufs[j], sws[j])
            return carry

        lax.fori_loop(0, nfired - nwaited, _wdrain, 0)

    def _zdrain(i, carry):
        pltpu.make_async_copy(x_hbm.at[pl.ds(0, C)], shared_z, sz).wait()
        return carry

    lax.fori_loop(0, MCH - mg, _zdrain, 0)


@functools.partial(jax.jit, static_argnums=())
def _compact(x_flat, valid_ids):
    mesh = plsc.VectorSubcoreMesh(core_axis_name="c", subcore_axis_name="s")
    f = pl.kernel(
        _compact_body,
        out_type=jax.ShapeDtypeStruct((B * S, D), jnp.float32),
        mesh=mesh,
        compiler_params=pltpu.CompilerParams(needs_layout_passes=False),
        scratch_types=[
            pltpu.VMEM((S,), jnp.int32),
            pltpu.VMEM((S,), jnp.int32),
            pltpu.VMEM((C, D), jnp.float32),
            pltpu.VMEM((C, D), jnp.float32),
            pltpu.VMEM((C, D), jnp.float32),
            pltpu.VMEM((C, D), jnp.float32),
            pltpu.VMEM_SHARED((C, D), jnp.float32),
            pltpu.SemaphoreType.DMA,
            pltpu.SemaphoreType.DMA,
            pltpu.SemaphoreType.DMA,
            pltpu.SemaphoreType.DMA,
            pltpu.SemaphoreType.DMA,
            pltpu.SemaphoreType.DMA,
            pltpu.SemaphoreType.DMA,
            pltpu.SemaphoreType.DMA,
            pltpu.SemaphoreType.DMA,
        ],
    )
    return f(x_flat, valid_ids)


def kernel(sequence_output, valid_ids):
    x_flat = sequence_output.reshape(B * S, D)
    out = _compact(x_flat, valid_ids)
    return out.reshape(B, S, D)
